# baseline (speedup 1.0000x reference)
import jax
import jax.numpy as jnp
from jax.experimental import pallas as pl
from jax.experimental.pallas import tpu as pltpu

_N = 20000
_NC = 21
_ROWS = 160
_LANES = 128
_NPAD = _ROWS * _LANES
_IMG_W = 800.0
_IMG_H = 800.0
_THR = 0.3
_STDS = (0.1, 0.1, 0.2, 0.2)
_BIG_I = 2 ** 30


def _nms_kernel(clss_ref, reg_ref, prop_ref,
                boxes_out, score_out, idx_out, keep_out,
                bx1_r, by1_r, bx2_r, by2_r, area_r, msk_r):
    maxl = clss_ref[0]
    for c in range(1, _NC):
        maxl = jnp.maximum(maxl, clss_ref[c])
    sumexp = jnp.exp(clss_ref[0] - maxl)
    for c in range(1, _NC):
        sumexp = sumexp + jnp.exp(clss_ref[c] - maxl)
    score = 1.0 / sumexp

    idx = jnp.full((_ROWS, _LANES), _NC - 1, jnp.int32)
    for c in range(_NC - 2, -1, -1):
        idx = jnp.where(clss_ref[c] == maxl, jnp.int32(c), idx)

    t0 = reg_ref[0]
    t1 = reg_ref[1]
    t2 = reg_ref[2]
    t3 = reg_ref[3]
    for c in range(1, _NC):
        m = idx == c
        t0 = jnp.where(m, reg_ref[4 * c + 0], t0)
        t1 = jnp.where(m, reg_ref[4 * c + 1], t1)
        t2 = jnp.where(m, reg_ref[4 * c + 2], t2)
        t3 = jnp.where(m, reg_ref[4 * c + 3], t3)

    x = prop_ref[0]
    y = prop_ref[1]
    w = prop_ref[2] - x
    h = prop_ref[3] - y
    px = x + w * (t0 * _STDS[0])
    py = y + h * (t1 * _STDS[1])
    pw = w * jnp.exp(t2 * _STDS[2])
    ph = h * jnp.exp(t3 * _STDS[3])
    bx1 = jnp.clip(px, 0.0, _IMG_W)
    by1 = jnp.clip(py, 0.0, _IMG_H)
    bx2 = jnp.clip(px + pw, 0.0, _IMG_W)
    by2 = jnp.clip(py + ph, 0.0, _IMG_H)
    area = (bx2 - bx1) * (by2 - by1)

    ridx = (jax.lax.broadcasted_iota(jnp.int32, (_ROWS, _LANES), 0) * _LANES
            + jax.lax.broadcasted_iota(jnp.int32, (_ROWS, _LANES), 1))
    alive0 = ((idx != 0) & (ridx < _N)).astype(jnp.float32)

    msk0 = jnp.where(alive0 > 0.0, score, -1.0)

    bx1_r[...] = bx1
    by1_r[...] = by1
    bx2_r[...] = bx2
    by2_r[...] = by2
    area_r[...] = area
    msk_r[...] = msk0
    score_out[...] = score
    idx_out[...] = idx

    lane = jax.lax.broadcasted_iota(jnp.int32, (1, _LANES), 1)
    _SHIFTS = (1, 2, 4, 8, 16, 32, 64)

    def _select(msk):
        colv = jnp.max(msk, axis=0, keepdims=True)
        coli = jnp.min(jnp.where(msk == colv, ridx, _BIG_I), axis=0,
                       keepdims=True)
        v, ix = colv, coli
        for s in _SHIFTS:
            rv = pltpu.roll(v, s, axis=1)
            ri = pltpu.roll(ix, s, axis=1)
            take = (rv > v) | ((rv == v) & (ri < ix))
            v = jnp.where(take, rv, v)
            ix = jnp.where(take, ri, ix)
        return v[0, 0], ix[0, 0]

    def cond(carry):
        mval, _ = carry
        return mval > 0.0

    def body(carry):
        _, midx = carry
        i0 = midx // _LANES
        oh = lane == (midx % _LANES)

        def bcast(ref):
            val = jnp.where(oh, ref[pl.ds(i0, 1), :], -1.0)
            for s in _SHIFTS:
                val = jnp.maximum(val, pltpu.roll(val, s, axis=1))
            return val

        ax1 = bcast(bx1_r)
        ay1 = bcast(by1_r)
        ax2 = bcast(bx2_r)
        ay2 = bcast(by2_r)
        aarea = (ax2 - ax1) * (ay2 - ay1)

        x1v = bx1_r[...]
        y1v = by1_r[...]
        x2v = bx2_r[...]
        y2v = by2_r[...]
        areav = area_r[...]
        inter = (jnp.clip(jnp.minimum(ax2, x2v) - jnp.maximum(ax1, x1v), 0.0)
                 * jnp.clip(jnp.minimum(ay2, y2v) - jnp.maximum(ay1, y1v), 0.0))
        iou = inter / (aarea + areav - inter + 1e-9)

        msk_l = msk_r[...]
        supp = (iou > _THR) & (msk_l > 0.0)
        sel1 = ridx == midx
        msk_new = jnp.where(sel1, -2.0, jnp.where(supp, -1.0, msk_l))
        msk_r[...] = msk_new
        return _select(msk_new)

    jax.lax.while_loop(cond, body, _select(msk0))

    keep = (msk_r[...] == -2.0).astype(jnp.float32)
    keep_i = keep.astype(jnp.int32)
    boxes_out[0] = bx1_r[...] * keep
    boxes_out[1] = by1_r[...] * keep
    boxes_out[2] = bx2_r[...] * keep
    boxes_out[3] = by2_r[...] * keep
    score_out[...] = score_out[...] * keep
    idx_out[...] = idx_out[...] * keep_i
    keep_out[...] = keep_i


def kernel(proposals, reg, clss):
    pad = _NPAD - _N
    clss_t = jnp.pad(clss, ((0, pad), (0, 0))).T.reshape(_NC, _ROWS, _LANES)
    reg_t = jnp.pad(reg, ((0, pad), (0, 0))).T.reshape(4 * _NC, _ROWS, _LANES)
    prop_t = jnp.pad(proposals, ((0, pad), (0, 0))).T.reshape(4, _ROWS, _LANES)

    boxes, score, idxs, keep = pl.pallas_call(
        _nms_kernel,
        out_shape=[
            jax.ShapeDtypeStruct((4, _ROWS, _LANES), jnp.float32),
            jax.ShapeDtypeStruct((_ROWS, _LANES), jnp.float32),
            jax.ShapeDtypeStruct((_ROWS, _LANES), jnp.int32),
            jax.ShapeDtypeStruct((_ROWS, _LANES), jnp.int32),
        ],
        scratch_shapes=[pltpu.VMEM((_ROWS, _LANES), jnp.float32)
                        for _ in range(6)],
    )(clss_t, reg_t, prop_t)

    refined = boxes.reshape(4, _NPAD).T[:_N]
    score_o = score.reshape(_NPAD)[:_N]
    idx_o = idxs.reshape(_NPAD)[:_N]
    keep_o = keep.reshape(_NPAD)[:_N].astype(bool)
    return (refined, score_o, idx_o, keep_o)


# speedup vs baseline: 2.2693x; 2.2693x over previous
import jax
import jax.numpy as jnp
from jax.experimental import pallas as pl
from jax.experimental.pallas import tpu as pltpu

_N = 20000
_NC = 21
_ROWS = 160
_LANES = 128
_NPAD = _ROWS * _LANES
_IMG_W = 800.0
_IMG_H = 800.0
_THR = 0.3
_STDS = (0.1, 0.1, 0.2, 0.2)
_BIG_I = 2 ** 30


def _nms_kernel(clss_ref, reg_ref, prop_ref,
                boxes_out, score_out, idx_out, keep_out,
                bx1_r, by1_r, bx2_r, by2_r, area_r, msk_r):
    maxl = clss_ref[0]
    for c in range(1, _NC):
        maxl = jnp.maximum(maxl, clss_ref[c])
    sumexp = jnp.exp(clss_ref[0] - maxl)
    for c in range(1, _NC):
        sumexp = sumexp + jnp.exp(clss_ref[c] - maxl)
    score = 1.0 / sumexp

    idx = jnp.full((_ROWS, _LANES), _NC - 1, jnp.int32)
    for c in range(_NC - 2, -1, -1):
        idx = jnp.where(clss_ref[c] == maxl, jnp.int32(c), idx)

    t0 = reg_ref[0]
    t1 = reg_ref[1]
    t2 = reg_ref[2]
    t3 = reg_ref[3]
    for c in range(1, _NC):
        m = idx == c
        t0 = jnp.where(m, reg_ref[4 * c + 0], t0)
        t1 = jnp.where(m, reg_ref[4 * c + 1], t1)
        t2 = jnp.where(m, reg_ref[4 * c + 2], t2)
        t3 = jnp.where(m, reg_ref[4 * c + 3], t3)

    x = prop_ref[0]
    y = prop_ref[1]
    w = prop_ref[2] - x
    h = prop_ref[3] - y
    px = x + w * (t0 * _STDS[0])
    py = y + h * (t1 * _STDS[1])
    pw = w * jnp.exp(t2 * _STDS[2])
    ph = h * jnp.exp(t3 * _STDS[3])
    bx1 = jnp.clip(px, 0.0, _IMG_W)
    by1 = jnp.clip(py, 0.0, _IMG_H)
    bx2 = jnp.clip(px + pw, 0.0, _IMG_W)
    by2 = jnp.clip(py + ph, 0.0, _IMG_H)
    area = (bx2 - bx1) * (by2 - by1)

    ridx = (jax.lax.broadcasted_iota(jnp.int32, (_ROWS, _LANES), 0) * _LANES
            + jax.lax.broadcasted_iota(jnp.int32, (_ROWS, _LANES), 1))
    alive0 = ((idx != 0) & (ridx < _N)).astype(jnp.float32)

    msk0 = jnp.where(alive0 > 0.0, score, -1.0)

    bx1_r[...] = bx1
    by1_r[...] = by1
    bx2_r[...] = bx2
    by2_r[...] = by2
    area_r[...] = area
    msk_r[...] = msk0
    score_out[...] = score
    idx_out[...] = idx

    lane = jax.lax.broadcasted_iota(jnp.int32, (1, _LANES), 1)

    def _select(msk):
        mval = jnp.max(msk)
        midx = jnp.min(jnp.where(msk == mval, ridx, _BIG_I))
        return mval, midx

    def cond(carry):
        mval, _, _ = carry
        return mval > 0.0

    def _extract(midx):
        i0 = midx // _LANES
        oh = lane == (midx % _LANES)
        x1m = jnp.max(jnp.where(oh, bx1_r[pl.ds(i0, 1), :], -1.0))
        y1m = jnp.max(jnp.where(oh, by1_r[pl.ds(i0, 1), :], -1.0))
        x2m = jnp.max(jnp.where(oh, bx2_r[pl.ds(i0, 1), :], -1.0))
        y2m = jnp.max(jnp.where(oh, by2_r[pl.ds(i0, 1), :], -1.0))
        return x1m, y1m, x2m, y2m

    def body(carry):
        _, midx1, msk_l = carry
        ax1, ay1, ax2, ay2 = _extract(midx1)
        aarea = (ax2 - ax1) * (ay2 - ay1)

        sel1 = ridx == midx1
        msk_ex = jnp.where(sel1, -1.0, msk_l)
        mval2 = jnp.max(msk_ex)
        midx2 = jnp.min(jnp.where(msk_ex == mval2, ridx, _BIG_I))
        bx1m, by1m, bx2m, by2m = _extract(midx2)
        barea = (bx2m - bx1m) * (by2m - by1m)

        cx1 = jnp.maximum(ax1, bx1m)
        cy1 = jnp.maximum(ay1, by1m)
        cx2 = jnp.minimum(ax2, bx2m)
        cy2 = jnp.minimum(ay2, by2m)
        cinter = jnp.clip(cx2 - cx1, 0.0) * jnp.clip(cy2 - cy1, 0.0)
        ciou = cinter / (aarea + barea - cinter + 1e-9)
        use2 = (mval2 > 0.0) & jnp.logical_not(ciou > _THR)

        x1v = bx1_r[...]
        y1v = by1_r[...]
        x2v = bx2_r[...]
        y2v = by2_r[...]
        areav = area_r[...]
        inter_a = (jnp.clip(jnp.minimum(ax2, x2v) - jnp.maximum(ax1, x1v), 0.0)
                   * jnp.clip(jnp.minimum(ay2, y2v) - jnp.maximum(ay1, y1v), 0.0))
        iou_a = inter_a / (aarea + areav - inter_a + 1e-9)
        inter_b = (jnp.clip(jnp.minimum(bx2m, x2v) - jnp.maximum(bx1m, x1v), 0.0)
                   * jnp.clip(jnp.minimum(by2m, y2v) - jnp.maximum(by1m, y1v), 0.0))
        iou_b = inter_b / (barea + areav - inter_b + 1e-9)

        supp = ((iou_a > _THR) | ((iou_b > _THR) & use2)) & (msk_ex > 0.0)
        sel2k = sel1 | (use2 & (ridx == midx2))
        msk_new = jnp.where(sel2k, -2.0, jnp.where(supp, -1.0, msk_l))

        mval_n, midx_n = _select(msk_new)
        return mval_n, midx_n, msk_new

    _, _, msk_fin = jax.lax.while_loop(
        cond, body, _select(msk0) + (msk0,))
    msk_r[...] = msk_fin

    keep = (msk_r[...] == -2.0).astype(jnp.float32)
    keep_i = keep.astype(jnp.int32)
    boxes_out[0] = bx1_r[...] * keep
    boxes_out[1] = by1_r[...] * keep
    boxes_out[2] = bx2_r[...] * keep
    boxes_out[3] = by2_r[...] * keep
    score_out[...] = score_out[...] * keep
    idx_out[...] = idx_out[...] * keep_i
    keep_out[...] = keep_i


def kernel(proposals, reg, clss):
    pad = _NPAD - _N
    clss_t = jnp.pad(clss, ((0, pad), (0, 0))).T.reshape(_NC, _ROWS, _LANES)
    reg_t = jnp.pad(reg, ((0, pad), (0, 0))).T.reshape(4 * _NC, _ROWS, _LANES)
    prop_t = jnp.pad(proposals, ((0, pad), (0, 0))).T.reshape(4, _ROWS, _LANES)

    boxes, score, idxs, keep = pl.pallas_call(
        _nms_kernel,
        out_shape=[
            jax.ShapeDtypeStruct((4, _ROWS, _LANES), jnp.float32),
            jax.ShapeDtypeStruct((_ROWS, _LANES), jnp.float32),
            jax.ShapeDtypeStruct((_ROWS, _LANES), jnp.int32),
            jax.ShapeDtypeStruct((_ROWS, _LANES), jnp.int32),
        ],
        scratch_shapes=[pltpu.VMEM((_ROWS, _LANES), jnp.float32)
                        for _ in range(6)],
    )(clss_t, reg_t, prop_t)

    refined = boxes.reshape(4, _NPAD).T[:_N]
    score_o = score.reshape(_NPAD)[:_N]
    idx_o = idxs.reshape(_NPAD)[:_N]
    keep_o = keep.reshape(_NPAD)[:_N].astype(bool)
    return (refined, score_o, idx_o, keep_o)


# speedup vs baseline: 2.3536x; 1.0372x over previous
import jax
import jax.numpy as jnp
from jax.experimental import pallas as pl
from jax.experimental.pallas import tpu as pltpu

_N = 20000
_NC = 21
_ROWS = 160
_LANES = 128
_NPAD = _ROWS * _LANES
_IMG_W = 800.0
_IMG_H = 800.0
_THR = 0.3
_STDS = (0.1, 0.1, 0.2, 0.2)
_BIG_I = 2 ** 30


def _nms_kernel(clss_ref, reg_ref, prop_ref,
                boxes_out, score_out, idx_out, keep_out,
                bx1_r, by1_r, bx2_r, by2_r, area_r, msk_r):
    maxl = clss_ref[0]
    for c in range(1, _NC):
        maxl = jnp.maximum(maxl, clss_ref[c])
    sumexp = jnp.exp(clss_ref[0] - maxl)
    for c in range(1, _NC):
        sumexp = sumexp + jnp.exp(clss_ref[c] - maxl)
    score = 1.0 / sumexp

    idx = jnp.full((_ROWS, _LANES), _NC - 1, jnp.int32)
    for c in range(_NC - 2, -1, -1):
        idx = jnp.where(clss_ref[c] == maxl, jnp.int32(c), idx)

    t0 = reg_ref[0]
    t1 = reg_ref[1]
    t2 = reg_ref[2]
    t3 = reg_ref[3]
    for c in range(1, _NC):
        m = idx == c
        t0 = jnp.where(m, reg_ref[4 * c + 0], t0)
        t1 = jnp.where(m, reg_ref[4 * c + 1], t1)
        t2 = jnp.where(m, reg_ref[4 * c + 2], t2)
        t3 = jnp.where(m, reg_ref[4 * c + 3], t3)

    x = prop_ref[0]
    y = prop_ref[1]
    w = prop_ref[2] - x
    h = prop_ref[3] - y
    px = x + w * (t0 * _STDS[0])
    py = y + h * (t1 * _STDS[1])
    pw = w * jnp.exp(t2 * _STDS[2])
    ph = h * jnp.exp(t3 * _STDS[3])
    bx1 = jnp.clip(px, 0.0, _IMG_W)
    by1 = jnp.clip(py, 0.0, _IMG_H)
    bx2 = jnp.clip(px + pw, 0.0, _IMG_W)
    by2 = jnp.clip(py + ph, 0.0, _IMG_H)
    area = (bx2 - bx1) * (by2 - by1)

    ridx = (jax.lax.broadcasted_iota(jnp.int32, (_ROWS, _LANES), 0) * _LANES
            + jax.lax.broadcasted_iota(jnp.int32, (_ROWS, _LANES), 1))
    alive0 = ((idx != 0) & (ridx < _N)).astype(jnp.float32)

    msk0 = jnp.where(alive0 > 0.0, score, -1.0)

    bx1_r[...] = bx1
    by1_r[...] = by1
    bx2_r[...] = bx2
    by2_r[...] = by2
    area_r[...] = area
    msk_r[...] = msk0
    score_out[...] = score
    idx_out[...] = idx

    lane = jax.lax.broadcasted_iota(jnp.int32, (1, _LANES), 1)

    def _select(msk):
        mval = jnp.max(msk)
        midx = jnp.min(jnp.where(msk == mval, ridx, _BIG_I))
        return mval, midx

    def cond(carry):
        mval, _ = carry
        return mval > 0.0

    def _extract(midx):
        i0 = midx // _LANES
        oh = lane == (midx % _LANES)
        x1m = jnp.max(jnp.where(oh, bx1_r[pl.ds(i0, 1), :], -1.0))
        y1m = jnp.max(jnp.where(oh, by1_r[pl.ds(i0, 1), :], -1.0))
        x2m = jnp.max(jnp.where(oh, bx2_r[pl.ds(i0, 1), :], -1.0))
        y2m = jnp.max(jnp.where(oh, by2_r[pl.ds(i0, 1), :], -1.0))
        return x1m, y1m, x2m, y2m

    def body(carry):
        _, midx1 = carry
        ax1, ay1, ax2, ay2 = _extract(midx1)
        aarea = (ax2 - ax1) * (ay2 - ay1)

        msk_l = msk_r[...]
        sel1 = ridx == midx1
        msk_ex = jnp.where(sel1, -1.0, msk_l)
        mval2 = jnp.max(msk_ex)
        midx2 = jnp.min(jnp.where(msk_ex == mval2, ridx, _BIG_I))
        bx1m, by1m, bx2m, by2m = _extract(midx2)
        barea = (bx2m - bx1m) * (by2m - by1m)

        cx1 = jnp.maximum(ax1, bx1m)
        cy1 = jnp.maximum(ay1, by1m)
        cx2 = jnp.minimum(ax2, bx2m)
        cy2 = jnp.minimum(ay2, by2m)
        cinter = jnp.clip(cx2 - cx1, 0.0) * jnp.clip(cy2 - cy1, 0.0)
        ciou = cinter / (aarea + barea - cinter + 1e-9)
        use2 = (mval2 > 0.0) & jnp.logical_not(ciou > _THR)

        x1v = bx1_r[...]
        y1v = by1_r[...]
        x2v = bx2_r[...]
        y2v = by2_r[...]
        areav = area_r[...]
        inter_a = (jnp.clip(jnp.minimum(ax2, x2v) - jnp.maximum(ax1, x1v), 0.0)
                   * jnp.clip(jnp.minimum(ay2, y2v) - jnp.maximum(ay1, y1v), 0.0))
        iou_a = inter_a / (aarea + areav - inter_a + 1e-9)
        inter_b = (jnp.clip(jnp.minimum(bx2m, x2v) - jnp.maximum(bx1m, x1v), 0.0)
                   * jnp.clip(jnp.minimum(by2m, y2v) - jnp.maximum(by1m, y1v), 0.0))
        iou_b = inter_b / (barea + areav - inter_b + 1e-9)

        supp = ((iou_a > _THR) | ((iou_b > _THR) & use2)) & (msk_ex > 0.0)
        msk_new = jnp.where(sel1, -2.0, jnp.where(supp, -1.0, msk_l))
        msk_r[...] = msk_new

        @pl.when(use2)
        def _():
            i2 = midx2 // _LANES
            oh2 = lane == (midx2 % _LANES)
            row = msk_r[pl.ds(i2, 1), :]
            msk_r[pl.ds(i2, 1), :] = jnp.where(oh2, -2.0, row)

        return _select(msk_new)

    jax.lax.while_loop(cond, body, _select(msk0))

    keep = (msk_r[...] == -2.0).astype(jnp.float32)
    keep_i = keep.astype(jnp.int32)
    boxes_out[0] = bx1_r[...] * keep
    boxes_out[1] = by1_r[...] * keep
    boxes_out[2] = bx2_r[...] * keep
    boxes_out[3] = by2_r[...] * keep
    score_out[...] = score_out[...] * keep
    idx_out[...] = idx_out[...] * keep_i
    keep_out[...] = keep_i


def kernel(proposals, reg, clss):
    pad = _NPAD - _N
    clss_t = jnp.pad(clss, ((0, pad), (0, 0))).T.reshape(_NC, _ROWS, _LANES)
    reg_t = jnp.pad(reg, ((0, pad), (0, 0))).T.reshape(4 * _NC, _ROWS, _LANES)
    prop_t = jnp.pad(proposals, ((0, pad), (0, 0))).T.reshape(4, _ROWS, _LANES)

    boxes, score, idxs, keep = pl.pallas_call(
        _nms_kernel,
        out_shape=[
            jax.ShapeDtypeStruct((4, _ROWS, _LANES), jnp.float32),
            jax.ShapeDtypeStruct((_ROWS, _LANES), jnp.float32),
            jax.ShapeDtypeStruct((_ROWS, _LANES), jnp.int32),
            jax.ShapeDtypeStruct((_ROWS, _LANES), jnp.int32),
        ],
        scratch_shapes=[pltpu.VMEM((_ROWS, _LANES), jnp.float32)
                        for _ in range(6)],
    )(clss_t, reg_t, prop_t)

    refined = boxes.reshape(4, _NPAD).T[:_N]
    score_o = score.reshape(_NPAD)[:_N]
    idx_o = idxs.reshape(_NPAD)[:_N]
    keep_o = keep.reshape(_NPAD)[:_N].astype(bool)
    return (refined, score_o, idx_o, keep_o)
